# Initial kernel scaffold; baseline (speedup 1.0000x reference)
import jax
import jax.numpy as jnp
from jax.experimental import pallas as pl


def kernel(xyz, new_xyz, features):
    raise NotImplementedError("write your pallas kernel here")


# speedup vs baseline: 343.1026x; 343.1026x over previous
import functools

import jax
import jax.numpy as jnp
from jax import lax
from jax.experimental import pallas as pl
from jax.experimental.pallas import tpu as pltpu
from jax.experimental.pallas import tpu_sc as plsc

RADIUS = 0.12
NSAMPLE = 32
BANDWIDTH = 0.1
RAD2 = RADIUS * RADIUS
NEG_INV_2BW2 = -1.0 / (2.0 * BANDWIDTH * BANDWIDTH)
DENS_SCALE = 1.0 / (NSAMPLE * 2.5 * BANDWIDTH)

B, N, M, C = 4, 4096, 1024, 128
MB = 256
NCHUNK = 512
NCHUNKS = N // NCHUNK


def _lt_strict():
    rows = lax.broadcasted_iota(jnp.int32, (NCHUNK, NCHUNK), 0)
    cols = lax.broadcasted_iota(jnp.int32, (NCHUNK, NCHUNK), 1)
    return (rows < cols).astype(jnp.bfloat16)


def _mask_d2(q, xt, qsq, xsq):
    cross = lax.dot_general(q, xt, (((1,), (0,)), ((), ())),
                            preferred_element_type=jnp.float32)
    return (qsq + xsq) - 2.0 * cross


def _density_body(q_ref, xt_ref, qsq_ref, xsq_ref, out_ref):
    q = q_ref[0]
    xt = xt_ref[0]
    qsq = qsq_ref[0]
    xsq = xsq_ref[0]
    d2m = _mask_d2(q, xt, qsq, xsq)
    within = (d2m < RAD2)
    mf = within.astype(jnp.float32)
    wb = within.astype(jnp.bfloat16)
    cross_a = (q[:, 0:1] * xt[0:1, :]
               + q[:, 1:2] * xt[1:2, :]
               + q[:, 2:3] * xt[2:3, :])
    d2a = (qsq + xsq) - 2.0 * cross_a
    w = jnp.exp(NEG_INV_2BW2 * d2a)

    lt = _lt_strict()
    carry = jnp.zeros((MB, 1), jnp.float32)
    sum_w = jnp.zeros((MB, 1), jnp.float32)
    first_w = jnp.zeros((MB, 1), jnp.float32)
    for c in range(NCHUNKS):
        sl = slice(c * NCHUNK, (c + 1) * NCHUNK)
        mf_c = mf[:, sl]
        rank = carry + lax.dot_general(wb[:, sl], lt, (((1,), (0,)), ((), ())),
                                       preferred_element_type=jnp.float32)
        sel = mf_c * (rank < float(NSAMPLE)).astype(jnp.float32)
        w_c = w[:, sl]
        sum_w = sum_w + jnp.sum(sel * w_c, axis=-1, keepdims=True)
        isfirst = mf_c * (rank < 0.5).astype(jnp.float32)
        first_w = first_w + jnp.sum(isfirst * w_c, axis=-1, keepdims=True)
        carry = carry + jnp.sum(mf_c, axis=-1, keepdims=True)
    cnt32 = jnp.minimum(carry, float(NSAMPLE))
    dens = (sum_w + (float(NSAMPLE) - cnt32) * first_w) * DENS_SCALE
    out_ref[0] = 1.0 / dens


def _density_call(q_pad, xt_pad, qsq3, xsq3):
    return pl.pallas_call(
        _density_body,
        grid=(B, N // MB),
        in_specs=[
            pl.BlockSpec((1, MB, 8), lambda b, m: (b, m, 0)),
            pl.BlockSpec((1, 8, N), lambda b, m: (b, 0, 0)),
            pl.BlockSpec((1, MB, 1), lambda b, m: (b, m, 0)),
            pl.BlockSpec((1, 1, N), lambda b, m: (b, 0, 0)),
        ],
        out_specs=pl.BlockSpec((1, MB, 1), lambda b, m: (b, m, 0)),
        out_shape=jax.ShapeDtypeStruct((B, N, 1), jnp.float32),
    )(q_pad, xt_pad, qsq3, xsq3)


def _ball_idx_body(q_ref, xt_ref, qsq_ref, xsq_ref, out_ref):
    q = q_ref[0]
    xt = xt_ref[0]
    qsq = qsq_ref[0]
    xsq = xsq_ref[0]
    d2m = _mask_d2(q, xt, qsq, xsq)
    within = (d2m < RAD2)
    mf = within.astype(jnp.float32)
    wb = within.astype(jnp.bfloat16)

    lt = _lt_strict()
    carry = jnp.zeros((MB, 1), jnp.float32)
    acc = [jnp.zeros((MB, 1), jnp.float32) for _ in range(NSAMPLE)]
    for c in range(NCHUNKS):
        sl = slice(c * NCHUNK, (c + 1) * NCHUNK)
        mf_c = mf[:, sl]
        rank = carry + lax.dot_general(wb[:, sl], lt, (((1,), (0,)), ((), ())),
                                       preferred_element_type=jnp.float32)
        tm = jnp.where(mf_c > 0.5, rank, -1.0)
        nval = (lax.broadcasted_iota(jnp.int32, (1, NCHUNK), 1).astype(jnp.float32)
                + float(c * NCHUNK))
        for s in range(NSAMPLE):
            hit = (tm == float(s)).astype(jnp.float32)
            acc[s] = acc[s] + jnp.sum(hit * nval, axis=-1, keepdims=True)
        carry = carry + jnp.sum(mf_c, axis=-1, keepdims=True)
    idx = jnp.concatenate(acc, axis=-1)
    slots = lax.broadcasted_iota(jnp.int32, (1, NSAMPLE), 1).astype(jnp.float32)
    idx = jnp.where(slots < carry, idx, idx[:, 0:1])
    out_ref[0] = idx.astype(jnp.int32)


def _ball_idx_call(q_pad, xt_pad, qsq3, xsq3):
    return pl.pallas_call(
        _ball_idx_body,
        grid=(B, M // MB),
        in_specs=[
            pl.BlockSpec((1, MB, 8), lambda b, m: (b, m, 0)),
            pl.BlockSpec((1, 8, N), lambda b, m: (b, 0, 0)),
            pl.BlockSpec((1, MB, 1), lambda b, m: (b, m, 0)),
            pl.BlockSpec((1, 1, N), lambda b, m: (b, 0, 0)),
        ],
        out_specs=pl.BlockSpec((1, MB, NSAMPLE), lambda b, m: (b, m, 0)),
        out_shape=jax.ShapeDtypeStruct((B, M, NSAMPLE), jnp.int32),
    )(q_pad, xt_pad, qsq3, xsq3)


CH = 132
TASKS = B * CH
MS = M * NSAMPLE
VECS = MS // 16


def _sc_gather_call(tables, idx_flat):
    info = plsc.get_sparse_core_info()
    nc, ns = info.num_cores, info.num_subcores
    nw = nc * ns
    base_tasks = TASKS // nw
    extra = TASKS - base_tasks * nw
    mesh = plsc.VectorSubcoreMesh(core_axis_name="c", subcore_axis_name="s")

    @functools.partial(
        pl.kernel,
        out_type=jax.ShapeDtypeStruct((TASKS, MS), jnp.float32),
        mesh=mesh,
        scratch_types=[
            pltpu.VMEM((MS,), jnp.int32),
            pltpu.VMEM((N,), jnp.float32),
            pltpu.VMEM((MS,), jnp.float32),
        ],
        compiler_params=pltpu.CompilerParams(needs_layout_passes=False),
    )
    def gather_kernel(tab_hbm, idx_hbm, out_hbm, idx_v, tab_v, out_v):
        wid = lax.axis_index("s") * nc + lax.axis_index("c")
        start = wid * base_tasks + jnp.minimum(wid, extra)
        ntask = base_tasks + (wid < extra).astype(jnp.int32)

        def do_task(k):
            j = start + k
            b = j // CH
            reload = jnp.logical_or(k == 0, j % CH == 0)

            @pl.when(reload)
            def _():
                pltpu.sync_copy(idx_hbm.at[b], idx_v)

            pltpu.sync_copy(tab_hbm.at[j], tab_v)

            def gbody(i, _):
                base = pl.multiple_of(i * 16, 16)
                vidx = idx_v[pl.ds(base, 16)]
                out_v[pl.ds(base, 16)] = plsc.load_gather(tab_v, [vidx])
                return 0

            lax.fori_loop(0, VECS, gbody, 0)
            pltpu.sync_copy(out_v, out_hbm.at[j])

        for k in range(base_tasks + 1):
            if k < base_tasks:
                do_task(jnp.int32(k))
            else:
                @pl.when(ntask > base_tasks)
                def _():
                    do_task(jnp.int32(k))

    return gather_kernel(tables, idx_flat)


def kernel(xyz, new_xyz, features):
    xt = jnp.transpose(xyz, (0, 2, 1))
    zpad5 = jnp.zeros((B, 5, N), jnp.float32)
    xt_pad = jnp.concatenate([xt, zpad5], axis=1)
    x_pad = jnp.concatenate(
        [xyz, jnp.zeros((B, N, 5), jnp.float32)], axis=-1)
    q_pad = jnp.concatenate(
        [new_xyz, jnp.zeros((B, M, 5), jnp.float32)], axis=-1)
    xsq = jnp.sum(xyz ** 2, axis=-1)
    qsq = jnp.sum(new_xyz ** 2, axis=-1)

    inv_density = _density_call(x_pad, xt_pad,
                                xsq[:, :, None], xsq[:, None, :])
    idx = _ball_idx_call(q_pad, xt_pad,
                         qsq[:, :, None], xsq[:, None, :])

    combined = jnp.concatenate(
        [xt, jnp.transpose(inv_density, (0, 2, 1)), features], axis=1
    )
    tables = combined.reshape(TASKS, N)
    idx_flat = idx.reshape(B, MS)

    out = _sc_gather_call(tables, idx_flat)
    return out.reshape(B, CH, M, NSAMPLE)


# speedup vs baseline: 352.0240x; 1.0260x over previous
import functools

import jax
import jax.numpy as jnp
from jax import lax
from jax.experimental import pallas as pl
from jax.experimental.pallas import tpu as pltpu
from jax.experimental.pallas import tpu_sc as plsc

RADIUS = 0.12
NSAMPLE = 32
BANDWIDTH = 0.1
RAD2 = RADIUS * RADIUS
NEG_INV_2BW2 = -1.0 / (2.0 * BANDWIDTH * BANDWIDTH)
DENS_SCALE = 1.0 / (NSAMPLE * 2.5 * BANDWIDTH)

B, N, M, C = 4, 4096, 1024, 128
MB = 256
NCHUNK = 512
NCHUNKS = N // NCHUNK


def _lt_strict():
    rows = lax.broadcasted_iota(jnp.int32, (NCHUNK, NCHUNK), 0)
    cols = lax.broadcasted_iota(jnp.int32, (NCHUNK, NCHUNK), 1)
    return (rows < cols).astype(jnp.bfloat16)


def _mask_d2(q, xt, qsq, xsq):
    cross = lax.dot_general(q, xt, (((1,), (0,)), ((), ())),
                            preferred_element_type=jnp.float32)
    return (qsq + xsq) - 2.0 * cross


def _density_body(q_ref, xt_ref, qsq_ref, xsq_ref, out_ref):
    q = q_ref[0]
    xt = xt_ref[0]
    qsq = qsq_ref[0]
    xsq = xsq_ref[0]
    d2m = _mask_d2(q, xt, qsq, xsq)
    within = (d2m < RAD2)
    mf = within.astype(jnp.float32)
    wb = within.astype(jnp.bfloat16)
    cross_a = (q[:, 0:1] * xt[0:1, :]
               + q[:, 1:2] * xt[1:2, :]
               + q[:, 2:3] * xt[2:3, :])
    d2a = (qsq + xsq) - 2.0 * cross_a
    w = jnp.exp(NEG_INV_2BW2 * d2a)

    lt = _lt_strict()
    carry = jnp.zeros((MB, 1), jnp.float32)
    sum_w = jnp.zeros((MB, 1), jnp.float32)
    first_w = jnp.zeros((MB, 1), jnp.float32)
    for c in range(NCHUNKS):
        sl = slice(c * NCHUNK, (c + 1) * NCHUNK)
        mf_c = mf[:, sl]
        rank = carry + lax.dot_general(wb[:, sl], lt, (((1,), (0,)), ((), ())),
                                       preferred_element_type=jnp.float32)
        sel = mf_c * (rank < float(NSAMPLE)).astype(jnp.float32)
        w_c = w[:, sl]
        sum_w = sum_w + jnp.sum(sel * w_c, axis=-1, keepdims=True)
        isfirst = mf_c * (rank < 0.5).astype(jnp.float32)
        first_w = first_w + jnp.sum(isfirst * w_c, axis=-1, keepdims=True)
        carry = carry + jnp.sum(mf_c, axis=-1, keepdims=True)
    cnt32 = jnp.minimum(carry, float(NSAMPLE))
    dens = (sum_w + (float(NSAMPLE) - cnt32) * first_w) * DENS_SCALE
    out_ref[0] = 1.0 / dens


def _density_call(q_pad, xt_pad, qsq3, xsq3):
    return pl.pallas_call(
        _density_body,
        grid=(B, N // MB),
        in_specs=[
            pl.BlockSpec((1, MB, 8), lambda b, m: (b, m, 0)),
            pl.BlockSpec((1, 8, N), lambda b, m: (b, 0, 0)),
            pl.BlockSpec((1, MB, 1), lambda b, m: (b, m, 0)),
            pl.BlockSpec((1, 1, N), lambda b, m: (b, 0, 0)),
        ],
        out_specs=pl.BlockSpec((1, MB, 1), lambda b, m: (b, m, 0)),
        out_shape=jax.ShapeDtypeStruct((B, N, 1), jnp.float32),
    )(q_pad, xt_pad, qsq3, xsq3)


def _ball_idx_body(q_ref, xt_ref, qsq_ref, xsq_ref, out_ref):
    q = q_ref[0]
    xt = xt_ref[0]
    qsq = qsq_ref[0]
    xsq = xsq_ref[0]
    d2m = _mask_d2(q, xt, qsq, xsq)
    within = (d2m < RAD2)
    mf = within.astype(jnp.float32)
    wb = within.astype(jnp.bfloat16)

    lt = _lt_strict()
    carry = jnp.zeros((MB, 1), jnp.float32)
    acc = [jnp.zeros((MB, 1), jnp.float32) for _ in range(NSAMPLE)]
    for c in range(NCHUNKS):
        sl = slice(c * NCHUNK, (c + 1) * NCHUNK)
        mf_c = mf[:, sl]
        rank = carry + lax.dot_general(wb[:, sl], lt, (((1,), (0,)), ((), ())),
                                       preferred_element_type=jnp.float32)
        tm = jnp.where(mf_c > 0.5, rank, -1.0)
        nval = (lax.broadcasted_iota(jnp.int32, (1, NCHUNK), 1).astype(jnp.float32)
                + float(c * NCHUNK))
        for s in range(NSAMPLE):
            acc[s] = acc[s] + jnp.sum(
                jnp.where(tm == float(s), nval, 0.0), axis=-1, keepdims=True)
        carry = carry + jnp.sum(mf_c, axis=-1, keepdims=True)
    idx = jnp.concatenate(acc, axis=-1)
    slots = lax.broadcasted_iota(jnp.int32, (1, NSAMPLE), 1).astype(jnp.float32)
    idx = jnp.where(slots < carry, idx, idx[:, 0:1])
    out_ref[0] = idx.astype(jnp.int32)


def _ball_idx_call(q_pad, xt_pad, qsq3, xsq3):
    return pl.pallas_call(
        _ball_idx_body,
        grid=(B, M // MB),
        in_specs=[
            pl.BlockSpec((1, MB, 8), lambda b, m: (b, m, 0)),
            pl.BlockSpec((1, 8, N), lambda b, m: (b, 0, 0)),
            pl.BlockSpec((1, MB, 1), lambda b, m: (b, m, 0)),
            pl.BlockSpec((1, 1, N), lambda b, m: (b, 0, 0)),
        ],
        out_specs=pl.BlockSpec((1, MB, NSAMPLE), lambda b, m: (b, m, 0)),
        out_shape=jax.ShapeDtypeStruct((B, M, NSAMPLE), jnp.int32),
    )(q_pad, xt_pad, qsq3, xsq3)


CH = 132
TASKS = B * CH
MS = M * NSAMPLE
HALF = MS // 2
VECS_H = HALF // 16


def _sc_gather_call(xt, dens, features, idx_flat):
    info = plsc.get_sparse_core_info()
    nc, ns = info.num_cores, info.num_subcores
    nw = nc * ns
    mesh = plsc.VectorSubcoreMesh(core_axis_name="c", subcore_axis_name="s")

    @functools.partial(
        pl.kernel,
        out_type=jax.ShapeDtypeStruct((TASKS, MS), jnp.float32),
        mesh=mesh,
        scratch_types=[
            pltpu.VMEM((MS,), jnp.int32),
            pltpu.VMEM((N,), jnp.float32),
            pltpu.VMEM((N,), jnp.float32),
            pltpu.VMEM((N,), jnp.float32),
            pltpu.VMEM((N,), jnp.float32),
            pltpu.VMEM((HALF,), jnp.float32),
            pltpu.VMEM((HALF,), jnp.float32),
            pltpu.VMEM((HALF,), jnp.float32),
            pltpu.VMEM((HALF,), jnp.float32),
        ],
        compiler_params=pltpu.CompilerParams(needs_layout_passes=False),
    )
    def gather_kernel(xt_hbm, dens_hbm, feat_hbm, idx_hbm, out_hbm,
                      idx_v, t0, t1, t2, t3, o0, o1, o2, o3):
        wid = lax.axis_index("s") * nc + lax.axis_index("c")
        tabs = (t0, t1, t2, t3)
        outs = (o0, o1, o2, o3)

        for b in range(B):
            pltpu.sync_copy(idx_hbm.at[b], idx_v)
            for t in range(4):
                pltpu.sync_copy(feat_hbm.at[b, 4 * wid + t], tabs[t])
            for h in range(2):
                def gbody(i, _):
                    base = pl.multiple_of(i * 16, 16)
                    vidx = idx_v[pl.ds(h * HALF + base, 16)]
                    for t in range(4):
                        outs[t][pl.ds(base, 16)] = plsc.load_gather(tabs[t], [vidx])
                    return 0

                lax.fori_loop(0, VECS_H, gbody, 0, unroll=2)
                for t in range(4):
                    row = b * CH + 4 + 4 * wid + t
                    pltpu.sync_copy(outs[t], out_hbm.at[row, pl.ds(h * HALF, HALF)])

            @pl.when(wid < 3)
            def _():
                pltpu.sync_copy(xt_hbm.at[b, wid], t0)

            @pl.when(wid == 3)
            def _():
                pltpu.sync_copy(dens_hbm.at[b], t0)

            @pl.when(wid < 4)
            def _():
                for h in range(2):
                    def sbody(i, _):
                        base = pl.multiple_of(i * 16, 16)
                        vidx = idx_v[pl.ds(h * HALF + base, 16)]
                        o0[pl.ds(base, 16)] = plsc.load_gather(t0, [vidx])
                        return 0

                    lax.fori_loop(0, VECS_H, sbody, 0, unroll=2)
                    pltpu.sync_copy(
                        o0, out_hbm.at[b * CH + wid, pl.ds(h * HALF, HALF)])

    return gather_kernel(xt, dens, features, idx_flat)


def kernel(xyz, new_xyz, features):
    xt = jnp.transpose(xyz, (0, 2, 1))
    zpad5 = jnp.zeros((B, 5, N), jnp.float32)
    xt_pad = jnp.concatenate([xt, zpad5], axis=1)
    x_pad = jnp.concatenate(
        [xyz, jnp.zeros((B, N, 5), jnp.float32)], axis=-1)
    q_pad = jnp.concatenate(
        [new_xyz, jnp.zeros((B, M, 5), jnp.float32)], axis=-1)
    xsq = jnp.sum(xyz ** 2, axis=-1)
    qsq = jnp.sum(new_xyz ** 2, axis=-1)

    inv_density = _density_call(x_pad, xt_pad,
                                xsq[:, :, None], xsq[:, None, :])
    idx = _ball_idx_call(q_pad, xt_pad,
                         qsq[:, :, None], xsq[:, None, :])

    idx_flat = idx.reshape(B, MS)
    dens = inv_density.reshape(B, N)

    out = _sc_gather_call(xt, dens, features, idx_flat)
    return out.reshape(B, CH, M, NSAMPLE)


# speedup vs baseline: 412.7165x; 1.1724x over previous
import functools

import jax
import jax.numpy as jnp
from jax import lax
from jax.experimental import pallas as pl
from jax.experimental.pallas import tpu as pltpu
from jax.experimental.pallas import tpu_sc as plsc

RADIUS = 0.12
NSAMPLE = 32
BANDWIDTH = 0.1
RAD2 = RADIUS * RADIUS
NEG_INV_2BW2 = -1.0 / (2.0 * BANDWIDTH * BANDWIDTH)
DENS_SCALE = 1.0 / (NSAMPLE * 2.5 * BANDWIDTH)

B, N, M, C = 4, 4096, 1024, 128
MB = 256
NCHUNK = 512
NCHUNKS = N // NCHUNK


def _lt_strict():
    rows = lax.broadcasted_iota(jnp.int32, (NCHUNK, NCHUNK), 0)
    cols = lax.broadcasted_iota(jnp.int32, (NCHUNK, NCHUNK), 1)
    return (rows < cols).astype(jnp.bfloat16)


def _mask_d2(q, xt, qsq, xsq):
    cross = lax.dot_general(q, xt, (((1,), (0,)), ((), ())),
                            preferred_element_type=jnp.float32)
    return (qsq + xsq) - 2.0 * cross


def _density_body(q_ref, xt_ref, qsq_ref, xsq_ref, ac_ref, xc_ref, out_ref):
    q = q_ref[0]
    xt = xt_ref[0]
    qsq = qsq_ref[0]
    xsq = xsq_ref[0]
    d2m = _mask_d2(q, xt, qsq, xsq)
    within = (d2m < RAD2)
    mf = within.astype(jnp.float32)
    wb = within.astype(jnp.bfloat16)
    corr = lax.dot_general(ac_ref[0], xc_ref[0], (((1,), (0,)), ((), ())),
                           preferred_element_type=jnp.float32)
    d2a = d2m - 2.0 * corr
    w = jnp.exp(NEG_INV_2BW2 * d2a)

    lt = _lt_strict()
    carry = jnp.zeros((MB, 1), jnp.float32)
    sum_w = jnp.zeros((MB, 1), jnp.float32)
    first_w = jnp.zeros((MB, 1), jnp.float32)
    for c in range(NCHUNKS):
        sl = slice(c * NCHUNK, (c + 1) * NCHUNK)
        mf_c = mf[:, sl]
        rank = carry + lax.dot_general(wb[:, sl], lt, (((1,), (0,)), ((), ())),
                                       preferred_element_type=jnp.float32)
        sel = mf_c * (rank < float(NSAMPLE)).astype(jnp.float32)
        w_c = w[:, sl]
        sum_w = sum_w + jnp.sum(sel * w_c, axis=-1, keepdims=True)
        isfirst = mf_c * (rank < 0.5).astype(jnp.float32)
        first_w = first_w + jnp.sum(isfirst * w_c, axis=-1, keepdims=True)
        carry = carry + jnp.sum(mf_c, axis=-1, keepdims=True)
    cnt32 = jnp.minimum(carry, float(NSAMPLE))
    dens = (sum_w + (float(NSAMPLE) - cnt32) * first_w) * DENS_SCALE
    out_ref[0] = 1.0 / dens


def _density_call(q_pad, xt_pad, qsq3, xsq3, ac, xc):
    return pl.pallas_call(
        _density_body,
        grid=(B, N // MB),
        in_specs=[
            pl.BlockSpec((1, MB, 8), lambda b, m: (b, m, 0)),
            pl.BlockSpec((1, 8, N), lambda b, m: (b, 0, 0)),
            pl.BlockSpec((1, MB, 1), lambda b, m: (b, m, 0)),
            pl.BlockSpec((1, 1, N), lambda b, m: (b, 0, 0)),
            pl.BlockSpec((1, MB, 16), lambda b, m: (b, m, 0)),
            pl.BlockSpec((1, 16, N), lambda b, m: (b, 0, 0)),
        ],
        out_specs=pl.BlockSpec((1, MB, 1), lambda b, m: (b, m, 0)),
        out_shape=jax.ShapeDtypeStruct((B, N, 1), jnp.float32),
    )(q_pad, xt_pad, qsq3, xsq3, ac, xc)


def _ball_idx_body(q_ref, xt_ref, qsq_ref, xsq_ref, out_ref):
    q = q_ref[0]
    xt = xt_ref[0]
    qsq = qsq_ref[0]
    xsq = xsq_ref[0]
    d2m = _mask_d2(q, xt, qsq, xsq)
    within = (d2m < RAD2)
    mf = within.astype(jnp.float32)
    wb = within.astype(jnp.bfloat16)

    lt = _lt_strict()
    carry = jnp.zeros((MB, 1), jnp.float32)
    acc = [jnp.zeros((MB, 1), jnp.float32) for _ in range(NSAMPLE)]
    for c in range(NCHUNKS):
        sl = slice(c * NCHUNK, (c + 1) * NCHUNK)
        mf_c = mf[:, sl]
        rank = carry + lax.dot_general(wb[:, sl], lt, (((1,), (0,)), ((), ())),
                                       preferred_element_type=jnp.float32)
        tm = jnp.where(mf_c > 0.5, rank, -1.0)
        nval = (lax.broadcasted_iota(jnp.int32, (1, NCHUNK), 1).astype(jnp.float32)
                + float(c * NCHUNK))
        for s in range(NSAMPLE):
            acc[s] = acc[s] + jnp.sum(
                jnp.where(tm == float(s), nval, 0.0), axis=-1, keepdims=True)
        carry = carry + jnp.sum(mf_c, axis=-1, keepdims=True)
    idx = jnp.concatenate(acc, axis=-1)
    slots = lax.broadcasted_iota(jnp.int32, (1, NSAMPLE), 1).astype(jnp.float32)
    idx = jnp.where(slots < carry, idx, idx[:, 0:1])
    out_ref[0] = idx.astype(jnp.int32)


def _ball_idx_call(q_pad, xt_pad, qsq3, xsq3):
    return pl.pallas_call(
        _ball_idx_body,
        grid=(B, M // MB),
        in_specs=[
            pl.BlockSpec((1, MB, 8), lambda b, m: (b, m, 0)),
            pl.BlockSpec((1, 8, N), lambda b, m: (b, 0, 0)),
            pl.BlockSpec((1, MB, 1), lambda b, m: (b, m, 0)),
            pl.BlockSpec((1, 1, N), lambda b, m: (b, 0, 0)),
        ],
        out_specs=pl.BlockSpec((1, MB, NSAMPLE), lambda b, m: (b, m, 0)),
        out_shape=jax.ShapeDtypeStruct((B, M, NSAMPLE), jnp.int32),
    )(q_pad, xt_pad, qsq3, xsq3)


CH = 132
MS = M * NSAMPLE
NQ = 8
QR = M // NQ


def _sc_gather_call(xt, dens, features, idx_flat):
    info = plsc.get_sparse_core_info()
    nc, ns = info.num_cores, info.num_subcores
    mesh = plsc.VectorSubcoreMesh(core_axis_name="c", subcore_axis_name="s")

    @functools.partial(
        pl.kernel,
        out_type=jax.ShapeDtypeStruct((B, CH, M, NSAMPLE), jnp.float32),
        mesh=mesh,
        scratch_types=[
            pltpu.VMEM((QR * NSAMPLE,), jnp.int32),
            [pltpu.VMEM((N,), jnp.float32) for _ in range(2)],
            [pltpu.VMEM((2, QR, NSAMPLE), jnp.float32) for _ in range(2)],
            pltpu.SemaphoreType.DMA,
        ],
        compiler_params=pltpu.CompilerParams(needs_layout_passes=False),
    )
    def gather_kernel(xt_hbm, dens_hbm, feat_hbm, idx_hbm, out_hbm,
                      idx_q, tabs, outs, sem):
        wid = lax.axis_index("s") * nc + lax.axis_index("c")

        def gather_quarter(p, chans):
            def gbody(r, _):
                off = pl.multiple_of(r * NSAMPLE, 16)
                vidx0 = idx_q[pl.ds(off, 16)]
                vidx1 = idx_q[pl.ds(off + 16, 16)]
                for t in range(chans):
                    outs[t][p, r, pl.ds(0, 16)] = plsc.load_gather(
                        tabs[t], [vidx0])
                    outs[t][p, r, pl.ds(16, 16)] = plsc.load_gather(
                        tabs[t], [vidx1])
                return 0

            lax.fori_loop(0, QR, gbody, 0, unroll=2)

        def run_rows(b, ch0, chans):
            pend = {}
            for q in range(NQ):
                p = q % 2
                pltpu.sync_copy(
                    idx_hbm.at[b, pl.ds(q * QR * NSAMPLE, QR * NSAMPLE)],
                    idx_q)
                if q >= 2:
                    for cp in pend.pop(q - 2):
                        cp.wait()
                gather_quarter(p, chans)
                cps = []
                for t in range(chans):
                    cp = pltpu.make_async_copy(
                        outs[t].at[p],
                        out_hbm.at[b, ch0 + t, pl.ds(q * QR, QR), :],
                        sem)
                    cp.start()
                    cps.append(cp)
                pend[q] = cps
            for q in sorted(pend):
                for cp in pend[q]:
                    cp.wait()

        for b in range(B):
            for g in range(2):
                for t in range(2):
                    pltpu.sync_copy(
                        feat_hbm.at[b, 4 * wid + 2 * g + t], tabs[t])
                run_rows(b, 4 + 4 * wid + 2 * g, 2)

            local = wid - 8 * b
            in_range = jnp.logical_and(local >= 0, local < 4)

            @pl.when(jnp.logical_and(in_range, local < 3))
            def _():
                pltpu.sync_copy(xt_hbm.at[b, local], tabs[0])

            @pl.when(local == 3)
            def _():
                pltpu.sync_copy(dens_hbm.at[b], tabs[0])

            @pl.when(in_range)
            def _():
                run_rows(b, local, 1)

    return gather_kernel(xt, dens, features, idx_flat)


def kernel(xyz, new_xyz, features):
    xt = jnp.transpose(xyz, (0, 2, 1))
    zpad5 = jnp.zeros((B, 5, N), jnp.float32)
    xt_pad = jnp.concatenate([xt, zpad5], axis=1)
    x_pad = jnp.concatenate(
        [xyz, jnp.zeros((B, N, 5), jnp.float32)], axis=-1)
    q_pad = jnp.concatenate(
        [new_xyz, jnp.zeros((B, M, 5), jnp.float32)], axis=-1)
    xsq = jnp.sum(xyz ** 2, axis=-1)
    qsq = jnp.sum(new_xyz ** 2, axis=-1)

    x_hi = xyz.astype(jnp.bfloat16).astype(jnp.float32)
    x_lo = xyz - x_hi
    zc10 = jnp.zeros((B, N, 10), jnp.float32)
    ac = jnp.concatenate([x_hi, x_lo, zc10], axis=-1)
    xc = jnp.transpose(
        jnp.concatenate([x_lo, x_hi, zc10], axis=-1), (0, 2, 1))

    inv_density = _density_call(x_pad, xt_pad,
                                xsq[:, :, None], xsq[:, None, :],
                                ac, xc)
    idx = _ball_idx_call(q_pad, xt_pad,
                         qsq[:, :, None], xsq[:, None, :])

    idx_flat = idx.reshape(B, MS)
    dens = inv_density.reshape(B, N)

    return _sc_gather_call(xt, dens, features, idx_flat)
